# baseline (speedup 1.0000x reference)
import functools
import math

import jax
import jax.numpy as jnp
from jax import lax
from jax.experimental import pallas as pl
from jax.experimental.pallas import tpu as pltpu
from jax.experimental.pallas import tpu_sc as plsc

_LOG_NC = math.log(1000)

_NC = 2
_NS = 16
_NW = _NC * _NS


def _stats_body(x_ref, tgt_ref, minidx_ref, sel_ref, entsum_ref,
                *, bblk, n_models, n_classes):
    i = pl.program_id(0)
    tgt = tgt_ref[...]
    crow = lax.broadcasted_iota(jnp.int32, (n_classes, bblk), 0)
    tmask = crow == tgt

    ce = []
    ent = []
    for m in range(n_models):
        x = x_ref[m]
        xmax = jnp.max(x, axis=0, keepdims=True)
        shifted = x - xmax
        s = jnp.sum(jnp.exp(shifted), axis=0, keepdims=True)
        logsm = shifted - jnp.log(s)
        lsm_t = jnp.sum(jnp.where(tmask, logsm, 0.0), axis=0, keepdims=True)
        ce.append(-lsm_t)
        mean_lsm = jnp.sum(logsm, axis=0, keepdims=True) / float(n_classes)
        ent.append((-_LOG_NC) - mean_lsm)

    sum_ent = ((ent[0] + ent[1]) + ent[2]) + ent[3]
    loss = [ce[m] + (sum_ent - ent[m]) for m in range(n_models)]

    best = loss[0]
    bidx = jnp.zeros((1, bblk), jnp.int32)
    for m in range(1, n_models):
        take = loss[m] < best
        best = jnp.where(take, loss[m], best)
        bidx = jnp.where(take, m, bidx)

    selv = ce[0] - ent[0]
    for m in range(1, n_models):
        selv = jnp.where(bidx == m, ce[m] - ent[m], selv)

    minidx_ref[...] = bidx

    @pl.when(i == 0)
    def _():
        sel_ref[...] = jnp.zeros((1, 1), jnp.float32)
        entsum_ref[...] = jnp.zeros((1, 1), jnp.float32)

    sel_ref[...] += jnp.sum(selv).reshape(1, 1)
    entsum_ref[...] += jnp.sum(sum_ent).reshape(1, 1)


def _tc_stats(pred_t, tgt_row, bblk):
    m, c, b = pred_t.shape
    nb = b // bblk
    body = functools.partial(_stats_body, bblk=bblk, n_models=m, n_classes=c)
    out_shapes = (
        jax.ShapeDtypeStruct((1, b), jnp.int32),
        jax.ShapeDtypeStruct((1, 1), jnp.float32),
        jax.ShapeDtypeStruct((1, 1), jnp.float32),
    )
    in_specs = [
        pl.BlockSpec((m, c, bblk), lambda i: (0, 0, i)),
        pl.BlockSpec((1, bblk), lambda i: (0, i)),
    ]
    out_specs = (
        pl.BlockSpec((1, bblk), lambda i: (0, i)),
        pl.BlockSpec((1, 1), lambda i: (0, 0)),
        pl.BlockSpec((1, 1), lambda i: (0, 0)),
    )
    return pl.pallas_call(
        body,
        grid=(nb,),
        in_specs=in_specs,
        out_specs=out_specs,
        out_shape=out_shapes,
    )(pred_t, tgt_row)


def _sc_select_body(x_hbm, idx_hbm, out_hbm, idx_v, xin, xout, sem0, sem1,
                    osem0, osem1, *, b_per_w, n_models, n_classes, kc):
    wid = lax.axis_index("s") * _NC + lax.axis_index("c")
    base = wid * b_per_w
    pltpu.sync_copy(idx_hbm.at[pl.ds(base, b_per_w)], idx_v)
    nchunks = n_classes // kc
    tail_rows = n_classes - nchunks * kc
    insems = (sem0, sem1)
    outsems = (osem0, osem1)

    def start_in(j, slot):
        c0 = j * kc
        for m in range(n_models):
            pltpu.async_copy(
                x_hbm.at[pl.ds(m * n_classes + c0, kc), pl.ds(base, b_per_w)],
                xin.at[slot].at[pl.ds(m * kc, kc)], insems[slot])

    def drain_in(slot):
        for m in range(n_models):
            pltpu.make_async_copy(
                x_hbm.at[pl.ds(0, kc), pl.ds(base, b_per_w)],
                xin.at[slot].at[pl.ds(0, kc)], insems[slot]).wait()

    def start_out(j, slot):
        c0 = j * kc
        pltpu.async_copy(
            xout.at[slot], out_hbm.at[pl.ds(c0, kc), pl.ds(base, b_per_w)],
            outsems[slot])

    def drain_out(slot):
        pltpu.make_async_copy(
            xout.at[slot], out_hbm.at[pl.ds(0, kc), pl.ds(base, b_per_w)],
            outsems[slot]).wait()

    def compute(slot, rows=None):
        def grp(g, carry2):
            del carry2
            mi = idx_v[pl.ds(g * 16, 16)]
            masks = [mi == m for m in range(1, n_models)]
            for r in range(kc if rows is None else rows):
                vals = xin[slot, r, pl.ds(g * 16, 16)]
                for m in range(1, n_models):
                    vals = jnp.where(
                        masks[m - 1],
                        xin[slot, m * kc + r, pl.ds(g * 16, 16)], vals)
                xout[slot, r, pl.ds(g * 16, 16)] = vals
            return 0

        lax.fori_loop(0, b_per_w // 16, grp, 0)

    start_in(0, 0)

    def pair(p, carry):
        del carry
        j0 = p * 2
        start_in(j0 + 1, 1)
        drain_in(0)

        @pl.when(p > 0)
        def _():
            drain_out(0)

        compute(0)
        start_out(j0, 0)
        @pl.when(j0 + 2 < nchunks)
        def _():
            start_in(j0 + 2, 0)

        drain_in(1)

        @pl.when(p > 0)
        def _():
            drain_out(1)

        compute(1)
        start_out(j0 + 1, 1)
        return 0

    npairs = nchunks // 2
    lax.fori_loop(0, npairs, pair, 0)
    if nchunks % 2:
        j = nchunks - 1
        drain_in(0)
        drain_out(0)
        compute(0)
        start_out(j, 0)
        drain_out(1)
        drain_out(0)
    else:
        drain_out(0)
        drain_out(1)

    if tail_rows:
        c0 = nchunks * kc
        copies = [
            pltpu.async_copy(
                x_hbm.at[pl.ds(m * n_classes + c0, tail_rows),
                         pl.ds(base, b_per_w)],
                xin.at[0].at[pl.ds(m * kc, tail_rows)], insems[0])
            for m in range(n_models)
        ]
        for cp in copies:
            cp.wait()
        compute(0, rows=tail_rows)
        pltpu.async_copy(
            xout.at[0].at[pl.ds(0, tail_rows)],
            out_hbm.at[pl.ds(c0, tail_rows), pl.ds(base, b_per_w)],
            outsems[0]).wait()


def _sc_select(x2d, min_idx, kc=24):
    mc, b = x2d.shape
    n_classes = 1000
    n_models = mc // n_classes
    b_per_w = b // _NW
    mesh = plsc.VectorSubcoreMesh(core_axis_name="c", subcore_axis_name="s")
    body = functools.partial(_sc_select_body, b_per_w=b_per_w,
                             n_models=n_models, n_classes=n_classes, kc=kc)
    return pl.kernel(
        body,
        out_type=jax.ShapeDtypeStruct((n_classes, b), jnp.float32),
        mesh=mesh,
        compiler_params=pltpu.CompilerParams(use_tc_tiling_on_sc=True),
        scratch_types=[
            pltpu.VMEM((b_per_w,), jnp.int32),
            pltpu.VMEM((2, n_models * kc, b_per_w), jnp.float32),
            pltpu.VMEM((2, kc, b_per_w), jnp.float32),
            pltpu.SemaphoreType.DMA,
            pltpu.SemaphoreType.DMA,
            pltpu.SemaphoreType.DMA,
            pltpu.SemaphoreType.DMA,
        ],
    )(x2d, min_idx)


def kernel(pred_logits_list, targets):
    m, b, c = pred_logits_list.shape
    pred_t = jnp.transpose(pred_logits_list, (0, 2, 1))
    tgt_row = targets.astype(jnp.int32).reshape(1, b)
    minidx_row, sel_sum, ent_sum = _tc_stats(pred_t, tgt_row, bblk=256)
    minidx = minidx_row.reshape(b)
    oracle_t = _sc_select(pred_t.reshape(m * c, b), minidx)
    oracle = jnp.transpose(oracle_t, (1, 0))
    new_loss = sel_sum[0, 0] / b + ent_sum[0, 0] / b
    return new_loss, oracle, minidx


# speedup vs baseline: 1.0356x; 1.0356x over previous
import functools
import math

import jax
import jax.numpy as jnp
from jax import lax
from jax.experimental import pallas as pl
from jax.experimental.pallas import tpu as pltpu
from jax.experimental.pallas import tpu_sc as plsc

_LOG_NC = math.log(1000)

_NC = 2
_NS = 16
_NW = _NC * _NS


def _stats_body(x_ref, tgt_ref, minidx_ref, sel_ref, entsum_ref,
                *, bblk, n_models, n_classes):
    i = pl.program_id(0)
    tgt = tgt_ref[...]
    crow = lax.broadcasted_iota(jnp.int32, (n_classes, bblk), 0)
    tmask = crow == tgt

    ce = []
    ent = []
    for m in range(n_models):
        x = x_ref[m]
        xmax = jnp.max(x, axis=0, keepdims=True)
        shifted = x - xmax
        s = jnp.sum(jnp.exp(shifted), axis=0, keepdims=True)
        logsm = shifted - jnp.log(s)
        lsm_t = jnp.sum(jnp.where(tmask, logsm, 0.0), axis=0, keepdims=True)
        ce.append(-lsm_t)
        mean_lsm = jnp.sum(logsm, axis=0, keepdims=True) / float(n_classes)
        ent.append((-_LOG_NC) - mean_lsm)

    sum_ent = ((ent[0] + ent[1]) + ent[2]) + ent[3]
    loss = [ce[m] + (sum_ent - ent[m]) for m in range(n_models)]

    best = loss[0]
    bidx = jnp.zeros((1, bblk), jnp.int32)
    for m in range(1, n_models):
        take = loss[m] < best
        best = jnp.where(take, loss[m], best)
        bidx = jnp.where(take, m, bidx)

    selv = ce[0] - ent[0]
    for m in range(1, n_models):
        selv = jnp.where(bidx == m, ce[m] - ent[m], selv)

    minidx_ref[...] = bidx

    @pl.when(i == 0)
    def _():
        sel_ref[...] = jnp.zeros((1, 1), jnp.float32)
        entsum_ref[...] = jnp.zeros((1, 1), jnp.float32)

    sel_ref[...] += jnp.sum(selv).reshape(1, 1)
    entsum_ref[...] += jnp.sum(sum_ent).reshape(1, 1)


def _tc_stats(pred_t, tgt_row, bblk):
    m, c, b = pred_t.shape
    nb = b // bblk
    body = functools.partial(_stats_body, bblk=bblk, n_models=m, n_classes=c)
    out_shapes = (
        jax.ShapeDtypeStruct((1, b), jnp.int32),
        jax.ShapeDtypeStruct((1, 1), jnp.float32),
        jax.ShapeDtypeStruct((1, 1), jnp.float32),
    )
    in_specs = [
        pl.BlockSpec((m, c, bblk), lambda i: (0, 0, i)),
        pl.BlockSpec((1, bblk), lambda i: (0, i)),
    ]
    out_specs = (
        pl.BlockSpec((1, bblk), lambda i: (0, i)),
        pl.BlockSpec((1, 1), lambda i: (0, 0)),
        pl.BlockSpec((1, 1), lambda i: (0, 0)),
    )
    return pl.pallas_call(
        body,
        grid=(nb,),
        in_specs=in_specs,
        out_specs=out_specs,
        out_shape=out_shapes,
    )(pred_t, tgt_row)


def _sc_select_body(x_hbm, idx_hbm, out_hbm, idx_v, xin, xout, sem0, sem1,
                    osem0, osem1, *, b_per_w, n_models, n_classes, kc):
    wid = lax.axis_index("s") * _NC + lax.axis_index("c")
    base = wid * b_per_w
    pltpu.sync_copy(idx_hbm.at[pl.ds(base, b_per_w)], idx_v)
    nchunks = n_classes // kc
    tail_rows = n_classes - nchunks * kc
    insems = (sem0, sem1)
    outsems = (osem0, osem1)

    def start_in(j, slot):
        c0 = j * kc
        for m in range(n_models):
            pltpu.async_copy(
                x_hbm.at[pl.ds(m * n_classes + c0, kc), pl.ds(base, b_per_w)],
                xin.at[slot].at[pl.ds(m * kc, kc)], insems[slot])

    def drain_in(slot):
        for m in range(n_models):
            pltpu.make_async_copy(
                x_hbm.at[pl.ds(0, kc), pl.ds(base, b_per_w)],
                xin.at[slot].at[pl.ds(0, kc)], insems[slot]).wait()

    def start_out(j, slot):
        c0 = j * kc
        pltpu.async_copy(
            xout.at[slot], out_hbm.at[pl.ds(c0, kc), pl.ds(base, b_per_w)],
            outsems[slot])

    def drain_out(slot):
        pltpu.make_async_copy(
            xout.at[slot], out_hbm.at[pl.ds(0, kc), pl.ds(base, b_per_w)],
            outsems[slot]).wait()

    def compute(slot, rows=None):
        def grp(g, carry2):
            del carry2
            mi = idx_v[pl.ds(g * 16, 16)]
            masks = [mi == m for m in range(1, n_models)]
            for r in range(kc if rows is None else rows):
                vals = xin[slot, r, pl.ds(g * 16, 16)]
                for m in range(1, n_models):
                    vals = jnp.where(
                        masks[m - 1],
                        xin[slot, m * kc + r, pl.ds(g * 16, 16)], vals)
                xout[slot, r, pl.ds(g * 16, 16)] = vals
            return 0

        lax.fori_loop(0, b_per_w // 16, grp, 0)

    start_in(0, 0)

    def pair(p, carry):
        del carry
        j0 = p * 2
        start_in(j0 + 1, 1)
        drain_in(0)

        @pl.when(p > 0)
        def _():
            drain_out(0)

        compute(0)
        start_out(j0, 0)
        @pl.when(j0 + 2 < nchunks)
        def _():
            start_in(j0 + 2, 0)

        drain_in(1)

        @pl.when(p > 0)
        def _():
            drain_out(1)

        compute(1)
        start_out(j0 + 1, 1)
        return 0

    npairs = nchunks // 2
    lax.fori_loop(0, npairs, pair, 0)
    if nchunks % 2:
        j = nchunks - 1
        drain_in(0)
        drain_out(0)
        compute(0)
        start_out(j, 0)
        drain_out(1)
        drain_out(0)
    else:
        drain_out(0)
        drain_out(1)

    if tail_rows:
        c0 = nchunks * kc
        copies = [
            pltpu.async_copy(
                x_hbm.at[pl.ds(m * n_classes + c0, tail_rows),
                         pl.ds(base, b_per_w)],
                xin.at[0].at[pl.ds(m * kc, tail_rows)], insems[0])
            for m in range(n_models)
        ]
        for cp in copies:
            cp.wait()
        compute(0, rows=tail_rows)
        pltpu.async_copy(
            xout.at[0].at[pl.ds(0, tail_rows)],
            out_hbm.at[pl.ds(c0, tail_rows), pl.ds(base, b_per_w)],
            outsems[0]).wait()


def _sc_select(x2d, min_idx, kc=16):
    mc, b = x2d.shape
    n_classes = 1000
    n_models = mc // n_classes
    b_per_w = b // _NW
    mesh = plsc.VectorSubcoreMesh(core_axis_name="c", subcore_axis_name="s")
    body = functools.partial(_sc_select_body, b_per_w=b_per_w,
                             n_models=n_models, n_classes=n_classes, kc=kc)
    return pl.kernel(
        body,
        out_type=jax.ShapeDtypeStruct((n_classes, b), jnp.float32),
        mesh=mesh,
        compiler_params=pltpu.CompilerParams(use_tc_tiling_on_sc=True),
        scratch_types=[
            pltpu.VMEM((b_per_w,), jnp.int32),
            pltpu.VMEM((2, n_models * kc, b_per_w), jnp.float32),
            pltpu.VMEM((2, kc, b_per_w), jnp.float32),
            pltpu.SemaphoreType.DMA,
            pltpu.SemaphoreType.DMA,
            pltpu.SemaphoreType.DMA,
            pltpu.SemaphoreType.DMA,
        ],
    )(x2d, min_idx)


def kernel(pred_logits_list, targets):
    m, b, c = pred_logits_list.shape
    pred_t = jnp.transpose(pred_logits_list, (0, 2, 1))
    tgt_row = targets.astype(jnp.int32).reshape(1, b)
    minidx_row, sel_sum, ent_sum = _tc_stats(pred_t, tgt_row, bblk=512)
    minidx = minidx_row.reshape(b)
    oracle_t = _sc_select(pred_t.reshape(m * c, b), minidx)
    oracle = jnp.transpose(oracle_t, (1, 0))
    new_loss = sel_sum[0, 0] / b + ent_sum[0, 0] / b
    return new_loss, oracle, minidx


# speedup vs baseline: 1.0595x; 1.0231x over previous
import functools
import math

import jax
import jax.numpy as jnp
from jax import lax
from jax.experimental import pallas as pl
from jax.experimental.pallas import tpu as pltpu
from jax.experimental.pallas import tpu_sc as plsc

_LOG_NC = math.log(1000)

_NC = 2
_NS = 16
_NW = _NC * _NS


def _stats_body(x_ref, tgt_ref, minidx_ref, sel_ref, entsum_ref,
                *, bblk, n_models, n_classes):
    i = pl.program_id(0)
    tgt = tgt_ref[...]
    crow = lax.broadcasted_iota(jnp.int32, (n_classes, bblk), 0)
    tmask = crow == tgt

    ce = []
    ent = []
    for m in range(n_models):
        x = x_ref[m]
        xmax = jnp.max(x, axis=0, keepdims=True)
        shifted = x - xmax
        s = jnp.sum(jnp.exp(shifted), axis=0, keepdims=True)
        logsm = shifted - jnp.log(s)
        lsm_t = jnp.sum(jnp.where(tmask, logsm, 0.0), axis=0, keepdims=True)
        ce.append(-lsm_t)
        mean_lsm = jnp.sum(logsm, axis=0, keepdims=True) / float(n_classes)
        ent.append((-_LOG_NC) - mean_lsm)

    sum_ent = ((ent[0] + ent[1]) + ent[2]) + ent[3]
    loss = [ce[m] + (sum_ent - ent[m]) for m in range(n_models)]

    best = loss[0]
    bidx = jnp.zeros((1, bblk), jnp.int32)
    for m in range(1, n_models):
        take = loss[m] < best
        best = jnp.where(take, loss[m], best)
        bidx = jnp.where(take, m, bidx)

    selv = ce[0] - ent[0]
    for m in range(1, n_models):
        selv = jnp.where(bidx == m, ce[m] - ent[m], selv)

    minidx_ref[...] = bidx.reshape(bblk)

    @pl.when(i == 0)
    def _():
        sel_ref[...] = jnp.zeros((1, 1), jnp.float32)
        entsum_ref[...] = jnp.zeros((1, 1), jnp.float32)

    sel_ref[...] += jnp.sum(selv).reshape(1, 1)
    entsum_ref[...] += jnp.sum(sum_ent).reshape(1, 1)


def _tc_stats(pred_t, tgt_row, bblk):
    m, c, b = pred_t.shape
    nb = b // bblk
    body = functools.partial(_stats_body, bblk=bblk, n_models=m, n_classes=c)
    out_shapes = (
        jax.ShapeDtypeStruct((b,), jnp.int32),
        jax.ShapeDtypeStruct((1, 1), jnp.float32),
        jax.ShapeDtypeStruct((1, 1), jnp.float32),
    )
    in_specs = [
        pl.BlockSpec((m, c, bblk), lambda i: (0, 0, i)),
        pl.BlockSpec((1, bblk), lambda i: (0, i)),
    ]
    out_specs = (
        pl.BlockSpec((bblk,), lambda i: (i,)),
        pl.BlockSpec((1, 1), lambda i: (0, 0)),
        pl.BlockSpec((1, 1), lambda i: (0, 0)),
    )
    return pl.pallas_call(
        body,
        grid=(nb,),
        in_specs=in_specs,
        out_specs=out_specs,
        out_shape=out_shapes,
    )(pred_t, tgt_row)


def _sc_select_body(x_hbm, idx_hbm, out_hbm, idx_v, xin, xout, sem0, sem1,
                    osem0, osem1, *, b_per_w, n_models, n_classes, kc):
    wid = lax.axis_index("s") * _NC + lax.axis_index("c")
    base = wid * b_per_w
    pltpu.sync_copy(idx_hbm.at[pl.ds(base, b_per_w)], idx_v)
    nchunks = n_classes // kc
    tail_rows = n_classes - nchunks * kc
    insems = (sem0, sem1)
    outsems = (osem0, osem1)

    def start_in(j, slot):
        c0 = j * kc
        for m in range(n_models):
            pltpu.async_copy(
                x_hbm.at[pl.ds(m * n_classes + c0, kc), pl.ds(base, b_per_w)],
                xin.at[slot].at[pl.ds(m * kc, kc)], insems[slot])

    def drain_in(slot):
        for m in range(n_models):
            pltpu.make_async_copy(
                x_hbm.at[pl.ds(0, kc), pl.ds(base, b_per_w)],
                xin.at[slot].at[pl.ds(0, kc)], insems[slot]).wait()

    def start_out(j, slot):
        c0 = j * kc
        pltpu.async_copy(
            xout.at[slot], out_hbm.at[pl.ds(c0, kc), pl.ds(base, b_per_w)],
            outsems[slot])

    def drain_out(slot):
        pltpu.make_async_copy(
            xout.at[slot], out_hbm.at[pl.ds(0, kc), pl.ds(base, b_per_w)],
            outsems[slot]).wait()

    def compute(slot, rows=None):
        def grp(g, carry2):
            del carry2
            mi = idx_v[pl.ds(g * 16, 16)]
            masks = [mi == m for m in range(1, n_models)]
            for r in range(kc if rows is None else rows):
                vals = xin[slot, r, pl.ds(g * 16, 16)]
                for m in range(1, n_models):
                    vals = jnp.where(
                        masks[m - 1],
                        xin[slot, m * kc + r, pl.ds(g * 16, 16)], vals)
                xout[slot, r, pl.ds(g * 16, 16)] = vals
            return 0

        lax.fori_loop(0, b_per_w // 16, grp, 0)

    start_in(0, 0)

    def pair(p, carry):
        del carry
        j0 = p * 2
        start_in(j0 + 1, 1)
        drain_in(0)

        @pl.when(p > 0)
        def _():
            drain_out(0)

        compute(0)
        start_out(j0, 0)
        @pl.when(j0 + 2 < nchunks)
        def _():
            start_in(j0 + 2, 0)

        drain_in(1)

        @pl.when(p > 0)
        def _():
            drain_out(1)

        compute(1)
        start_out(j0 + 1, 1)
        return 0

    npairs = nchunks // 2
    lax.fori_loop(0, npairs, pair, 0)
    if nchunks % 2:
        j = nchunks - 1
        drain_in(0)
        drain_out(0)
        compute(0)
        start_out(j, 0)
        drain_out(1)
        drain_out(0)
    else:
        drain_out(0)
        drain_out(1)

    if tail_rows:
        c0 = nchunks * kc
        copies = [
            pltpu.async_copy(
                x_hbm.at[pl.ds(m * n_classes + c0, tail_rows),
                         pl.ds(base, b_per_w)],
                xin.at[0].at[pl.ds(m * kc, tail_rows)], insems[0])
            for m in range(n_models)
        ]
        for cp in copies:
            cp.wait()
        compute(0, rows=tail_rows)
        pltpu.async_copy(
            xout.at[0].at[pl.ds(0, tail_rows)],
            out_hbm.at[pl.ds(c0, tail_rows), pl.ds(base, b_per_w)],
            outsems[0]).wait()


def _sc_select(x2d, min_idx, kc=24):
    mc, b = x2d.shape
    n_classes = 1000
    n_models = mc // n_classes
    b_per_w = b // _NW
    mesh = plsc.VectorSubcoreMesh(core_axis_name="c", subcore_axis_name="s")
    body = functools.partial(_sc_select_body, b_per_w=b_per_w,
                             n_models=n_models, n_classes=n_classes, kc=kc)
    return pl.kernel(
        body,
        out_type=jax.ShapeDtypeStruct((n_classes, b), jnp.float32),
        mesh=mesh,
        compiler_params=pltpu.CompilerParams(use_tc_tiling_on_sc=True),
        scratch_types=[
            pltpu.VMEM((b_per_w,), jnp.int32),
            pltpu.VMEM((2, n_models * kc, b_per_w), jnp.float32),
            pltpu.VMEM((2, kc, b_per_w), jnp.float32),
            pltpu.SemaphoreType.DMA,
            pltpu.SemaphoreType.DMA,
            pltpu.SemaphoreType.DMA,
            pltpu.SemaphoreType.DMA,
        ],
    )(x2d, min_idx)


def kernel(pred_logits_list, targets):
    m, b, c = pred_logits_list.shape
    pred_t = jnp.transpose(pred_logits_list, (0, 2, 1))
    tgt_row = targets.astype(jnp.int32).reshape(1, b)
    minidx, sel_sum, ent_sum = _tc_stats(pred_t, tgt_row, bblk=512)
    oracle_t = _sc_select(pred_t.reshape(m * c, b), minidx)
    oracle = jnp.transpose(oracle_t, (1, 0))
    new_loss = sel_sum[0, 0] / b + ent_sum[0, 0] / b
    return new_loss, oracle, minidx
